# baseline (speedup 1.0000x reference)
import functools
import math

import jax
import jax.numpy as jnp
from jax import lax
from jax.experimental import pallas as pl
from jax.experimental.pallas import tpu as pltpu
from jax.experimental.pallas import tpu_sc as plsc

_THRESH = 0.8
_MIN_KEPT = 100000
_ROWS = 256
_SUB = 8
_L08 = -math.log(_THRESH)


def _make_fused_body(write_loss):
    def body(lthr_ref, score_ref, tgt_ref, *out_refs):
        sum_ref, cnt_ref = out_refs[0], out_refs[1]
        i = pl.program_id(0)
        j = pl.program_id(1)
        c = score_ref.shape[1]
        w = score_ref.shape[3]
        thr = lthr_ref[i]
        acc = jnp.zeros((_SUB, w), jnp.float32)
        cnt = jnp.zeros((_SUB, w), jnp.float32)
        for rt in range(_ROWS // _SUB):
            rows = pl.ds(rt * _SUB, _SUB)
            t = tgt_ref[0, rows, :]
            m = score_ref[0, 0, rows, :]
            for cc in range(1, c):
                m = jnp.maximum(m, score_ref[0, cc, rows, :])
            s = jnp.zeros((_SUB, w), jnp.float32)
            xt = jnp.zeros((_SUB, w), jnp.float32)
            for cc in range(c):
                d = score_ref[0, cc, rows, :] - m
                s = s + jnp.exp(d)
                xt = xt + jnp.where(t == cc, d, 0.0)
            loss = jnp.log(s) - xt
            if write_loss:
                out_refs[2][0, rows, :] = loss
            keep = loss > thr
            acc = acc + jnp.where(keep, loss, 0.0)
            cnt = cnt + keep.astype(jnp.float32)

        @pl.when(j == 0)
        def _():
            sum_ref[i] = 0.0
            cnt_ref[i] = 0.0

        sum_ref[i] += jnp.sum(acc)
        cnt_ref[i] += jnp.sum(cnt)

    return body


def _make_sc_select(b, h, w, k_sel):
    chunk = 65536
    nchunk = h * w // chunk
    mesh = plsc.VectorSubcoreMesh(core_axis_name="c", subcore_axis_name="s")

    @functools.partial(
        pl.kernel,
        mesh=mesh,
        out_type=jax.ShapeDtypeStruct((b, 16), jnp.float32),
        scratch_types=[
            pltpu.VMEM((chunk,), jnp.float32),
            pltpu.VMEM((16,), jnp.float32),
        ],
    )
    def sel(loss_hbm, out_hbm, slab, vec16):
        wid = lax.axis_index("s")
        core = lax.axis_index("c")
        kf16 = jnp.full((16,), float(k_sel), jnp.float32)
        dn = lax.GatherDimensionNumbers(
            offset_dims=(), collapsed_slice_dims=(0,), start_index_map=(0,))
        lane = lax.iota(jnp.int32, 16)

        for img in range(b):

            @pl.when(jnp.logical_and(core == 0, wid == img))
            def _(img=img):
                def search_step(_, carry):
                    lo16, bit16 = carry
                    mid16 = lo16 + bit16
                    tot = jnp.zeros((16,), jnp.float32)
                    for cc in range(nchunk):
                        pltpu.sync_copy(
                            loss_hbm.at[pl.ds(img * h * w + cc * chunk,
                                              chunk)], slab)

                        def lane_body(kk, acc_k):
                            v = slab[pl.ds(kk * 16, 16)]
                            bits = lax.bitcast_convert_type(v, jnp.int32)
                            return acc_k + jnp.where(bits < mid16, 1.0, 0.0)

                        tot = lax.fori_loop(0, chunk // 16, lane_body, tot)
                    for stride in (1, 2, 4, 8):
                        g = lax.gather(
                            tot, (lane ^ stride).reshape(16, 1), dn, (1,),
                            mode=lax.GatherScatterMode.PROMISE_IN_BOUNDS)
                        tot = tot + g
                    lo16 = jnp.where(tot < kf16, mid16, lo16)
                    return lo16, lax.shift_right_logical(
                        bit16, jnp.full((16,), 1, jnp.int32))

                lo16, _ = lax.fori_loop(
                    0, 31, search_step,
                    (jnp.zeros((16,), jnp.int32),
                     jnp.full((16,), 1 << 30, jnp.int32)))
                vec16[...] = lax.bitcast_convert_type(lo16, jnp.float32)
                pltpu.sync_copy(vec16, out_hbm.at[img])

    return sel


@jax.jit
def kernel(score, target):
    b, c, h, w = score.shape
    target = target.astype(jnp.int32)
    nblk = h // _ROWS
    k0 = min(_MIN_KEPT, h * w - 1)

    def fused_call(write_loss):
        out_shape = [
            jax.ShapeDtypeStruct((b,), jnp.float32),
            jax.ShapeDtypeStruct((b,), jnp.float32),
        ]
        out_specs = [
            pl.BlockSpec(memory_space=pltpu.SMEM),
            pl.BlockSpec(memory_space=pltpu.SMEM),
        ]
        if write_loss:
            out_shape.append(jax.ShapeDtypeStruct((b, h, w), jnp.float32))
            out_specs.append(pl.BlockSpec((1, _ROWS, w), lambda i, j: (i, j, 0)))
        return pl.pallas_call(
            _make_fused_body(write_loss),
            grid=(b, nblk),
            in_specs=[
                pl.BlockSpec(memory_space=pltpu.SMEM),
                pl.BlockSpec((1, c, _ROWS, w), lambda i, j: (i, 0, j, 0)),
                pl.BlockSpec((1, _ROWS, w), lambda i, j: (i, j, 0)),
            ],
            out_specs=out_specs,
            out_shape=out_shape,
        )

    select = _make_sc_select(b, h, w, h * w - k0)

    thr0 = jnp.full((b,), _L08, jnp.float32)
    sums, cnts = fused_call(False)(thr0, score, target)

    def rare_path(_):
        _, _, loss = fused_call(True)(thr0, score, target)
        lk = select(loss.reshape(-1))[:, 0]
        s2, c2 = fused_call(False)(jnp.minimum(lk, _L08), score, target)
        return s2, c2

    sums, cnts = lax.cond(jnp.any(cnts < float(k0 + 1)), rare_path,
                          lambda _: (sums, cnts), operand=None)
    return jnp.sum(sums / jnp.maximum(cnts, 1.0)) / b
